# baseline (speedup 1.0000x reference)
import functools

import jax
import jax.numpy as jnp
from jax import lax
from jax.experimental import pallas as pl
from jax.experimental.pallas import tpu as pltpu
from jax.experimental.pallas import tpu_sc as plsc

F = 26
B = 4096
D = 32

NC = 2
NS = 16
NW = NC * NS

CHUNK = 128


def _build_gather(total, d):
  assert total % NW == 0
  per_w = total // NW
  assert per_w % CHUNK == 0
  n_chunks = per_w // CHUNK

  mesh = plsc.VectorSubcoreMesh(core_axis_name="c", subcore_axis_name="s")

  @functools.partial(
      pl.kernel,
      out_type=jax.ShapeDtypeStruct((total, d), jnp.float32),
      mesh=mesh,
      scratch_types=[
          pltpu.VMEM((n_chunks, CHUNK), jnp.int32),
          pltpu.VMEM((per_w, d), jnp.float32),
          pltpu.SemaphoreType.DMA,
      ],
      compiler_params=pltpu.CompilerParams(use_tc_tiling_on_sc=False),
  )
  def gather_kernel(table_hbm, idx_hbm, out_hbm, idx_v, rows_v, sem):
    wid = lax.axis_index("s") * NC + lax.axis_index("c")
    base = wid * per_w
    pltpu.sync_copy(idx_hbm.at[wid], idx_v)
    copies = []
    for j in range(n_chunks):
      copies.append(
          pltpu.async_copy(
              table_hbm.at[idx_v.at[j]],
              rows_v.at[pl.ds(j * CHUNK, CHUNK)],
              sem,
          )
      )
    for c in copies:
      c.wait()
    pltpu.sync_copy(rows_v, out_hbm.at[pl.ds(base, per_w)])

  return gather_kernel


_GATHER = _build_gather(F * B, D)


def kernel(table, values, lengths):
  idx = values.reshape(NW, (F * B) // NW // CHUNK, CHUNK)
  rows = _GATHER(table, idx)
  split_embeddings = rows.reshape(F, B, D)
  split_lengths = lengths.reshape(F, B)
  reduce_lengths = split_lengths.sum(axis=1)
  offsets = jnp.concatenate([
      jnp.zeros((1,), dtype=reduce_lengths.dtype),
      jnp.cumsum(reduce_lengths),
  ])
  return split_embeddings, split_lengths, offsets


# speedup vs baseline: 1.0368x; 1.0368x over previous
import functools

import jax
import jax.numpy as jnp
from jax import lax
from jax.experimental import pallas as pl
from jax.experimental.pallas import tpu as pltpu
from jax.experimental.pallas import tpu_sc as plsc

F = 26
B = 4096
D = 32
V = 1000000

NC = 2
NS = 16
NW = NC * NS

CHUNK = 128

VBLK = 2048
GRID = -(-V // VBLK)


def _transpose_body(in_ref, out_ref):
  x = in_ref[...].reshape(D, VBLK)
  y = x.T.reshape(VBLK // 4, 4, D)
  for a in range(4):
    out_ref[:, a * 32:(a + 1) * 32] = y[:, a, :]


_TRANSPOSE = pl.pallas_call(
    _transpose_body,
    grid=(GRID,),
    in_specs=[pl.BlockSpec((4, 8, VBLK), lambda j: (0, 0, j))],
    out_specs=pl.BlockSpec((VBLK * D // 128, 128), lambda j: (j, 0)),
    out_shape=jax.ShapeDtypeStruct((V * D // 128, 128), jnp.float32),
)


def _build_gather(total, d):
  per_w = total // NW
  n_chunks = per_w // CHUNK

  mesh = plsc.VectorSubcoreMesh(core_axis_name="c", subcore_axis_name="s")

  @functools.partial(
      pl.kernel,
      out_type=jax.ShapeDtypeStruct((total, d), jnp.float32),
      mesh=mesh,
      scratch_types=[
          pltpu.VMEM((n_chunks, CHUNK), jnp.int32),
          pltpu.VMEM((per_w, d), jnp.float32),
          pltpu.SemaphoreType.DMA,
      ],
      compiler_params=pltpu.CompilerParams(use_tc_tiling_on_sc=False),
  )
  def gather_kernel(table_hbm, idx_hbm, out_hbm, idx_v, rows_v, sem):
    wid = lax.axis_index("s") * NC + lax.axis_index("c")
    base = wid * per_w
    pltpu.sync_copy(idx_hbm.at[wid], idx_v)
    copies = []
    for j in range(n_chunks):
      copies.append(
          pltpu.async_copy(
              table_hbm.at[idx_v.at[j]],
              rows_v.at[pl.ds(j * CHUNK, CHUNK)],
              sem,
          )
      )
    for c in copies:
      c.wait()
    pltpu.sync_copy(rows_v, out_hbm.at[pl.ds(base, per_w)])

  return gather_kernel


_GATHER = _build_gather(F * B, D)


def kernel(table, values, lengths):
  tab3 = table.T.reshape(4, 8, V)
  tablin = _TRANSPOSE(tab3)
  tab_flat = tablin.reshape(V, D)
  idx = values.reshape(NW, (F * B) // NW // CHUNK, CHUNK)
  rows = _GATHER(tab_flat, idx)
  split_embeddings = rows.reshape(F, B, D)
  split_lengths = lengths.reshape(F, B)
  reduce_lengths = split_lengths.sum(axis=1)
  offsets = jnp.concatenate([
      jnp.zeros((1,), dtype=reduce_lengths.dtype),
      jnp.cumsum(reduce_lengths),
  ])
  return split_embeddings, split_lengths, offsets


# speedup vs baseline: 2.6477x; 2.5537x over previous
import functools

import jax
import jax.numpy as jnp
from jax import lax
from jax.experimental import pallas as pl
from jax.experimental.pallas import tpu as pltpu
from jax.experimental.pallas import tpu_sc as plsc

F = 26
B = 4096
D = 32
V = 1000000

NC = 2
NS = 16
NW = NC * NS

CHUNK = 128

VBLK = 8192
GRID = -(-V // VBLK)


def _transpose_body(in_ref, out_ref):
  x = in_ref[...].reshape(D, VBLK)
  for c in range(VBLK // 512):
    xs = jnp.concatenate(
        [x[:, 512 * c + 128 * a:512 * c + 128 * (a + 1)] for a in range(4)],
        axis=0,
    )
    out_ref[128 * c:128 * (c + 1), :] = xs.T


_TRANSPOSE = pl.pallas_call(
    _transpose_body,
    grid=(GRID,),
    in_specs=[pl.BlockSpec((4, 8, VBLK), lambda j: (0, 0, j))],
    out_specs=pl.BlockSpec((VBLK * D // 128, 128), lambda j: (j, 0)),
    out_shape=jax.ShapeDtypeStruct((GRID * VBLK * D // 128, 128), jnp.float32),
)


def _permuted_rows(values):
  v = values
  return (
      (v & ~(VBLK - 1))
      + ((v >> 9) & (VBLK // 512 - 1)) * 512
      + ((v & 127) << 2)
      + ((v >> 7) & 3)
  )


def _build_gather(total, d):
  per_w = total // NW
  n_chunks = per_w // CHUNK

  mesh = plsc.VectorSubcoreMesh(core_axis_name="c", subcore_axis_name="s")

  @functools.partial(
      pl.kernel,
      out_type=jax.ShapeDtypeStruct((total, d), jnp.float32),
      mesh=mesh,
      scratch_types=[
          pltpu.VMEM((n_chunks, CHUNK), jnp.int32),
          pltpu.VMEM((per_w, d), jnp.float32),
          pltpu.SemaphoreType.DMA,
      ],
      compiler_params=pltpu.CompilerParams(use_tc_tiling_on_sc=False),
  )
  def gather_kernel(table_hbm, idx_hbm, out_hbm, idx_v, rows_v, sem):
    wid = lax.axis_index("s") * NC + lax.axis_index("c")
    base = wid * per_w
    pltpu.sync_copy(idx_hbm.at[wid], idx_v)
    copies = []
    for j in range(n_chunks):
      copies.append(
          pltpu.async_copy(
              table_hbm.at[idx_v.at[j]],
              rows_v.at[pl.ds(j * CHUNK, CHUNK)],
              sem,
          )
      )
    for c in copies:
      c.wait()
    pltpu.sync_copy(rows_v, out_hbm.at[pl.ds(base, per_w)])

  return gather_kernel


_GATHER = _build_gather(F * B, D)


def kernel(table, values, lengths):
  tab3 = table.T.reshape(4, 8, V)
  tablin = _TRANSPOSE(tab3)
  tab_flat = tablin.reshape(GRID * VBLK, D)
  idx = _permuted_rows(values).reshape(NW, (F * B) // NW // CHUNK, CHUNK)
  rows = _GATHER(tab_flat, idx)
  split_embeddings = rows.reshape(F, B, D)
  split_lengths = lengths.reshape(F, B)
  reduce_lengths = split_lengths.sum(axis=1)
  offsets = jnp.concatenate([
      jnp.zeros((1,), dtype=reduce_lengths.dtype),
      jnp.cumsum(reduce_lengths),
  ])
  return split_embeddings, split_lengths, offsets
